# baseline (speedup 1.0000x reference)
import jax
import jax.numpy as jnp
from jax import lax
from jax.experimental import pallas as pl
from jax.experimental.pallas import tpu as pltpu
from jax.experimental.pallas import tpu_sc as plsc

N = 10000
E = 320000
IN_FEATS = 128
H = 256
NUM_CLASSES = 64
HH = H // 2

NT = 16
NSC = 2
EPT = E // NT
K = 40
CB = 25
NBLK = EPT // (CB * K)
NPAD = 10240
ROWS_PT = NPAD // NT
ZROWS = 8

_MESH = plsc.VectorSubcoreMesh(core_axis_name="c", subcore_axis_name="s")


NCHUNK = EPT // K
NB = 5
PF = 2
SOFF = 4


def _agg_body(x2_hbm, srcb_hbm, dstb_hbm, out_hbm, sstage, dstage,
              gbuf, zbuf, spmem_acc, semg, sems):
    c = lax.axis_index("c")
    s = lax.axis_index("s")
    zeros16 = jnp.zeros((16,), jnp.float32)

    def zrow(i, _):
        for j in range(HH // 16):
            zbuf[i, pl.ds(j * 16, 16)] = zeros16
        return 0
    lax.fori_loop(0, ZROWS, zrow, 0)

    def zcp(k, _):
        pltpu.sync_copy(zbuf, spmem_acc.at[pl.ds(s * ROWS_PT + k * ZROWS, ZROWS)])
        return 0
    lax.fori_loop(0, ROWS_PT // ZROWS, zcp, 0)

    pltpu.sync_copy(srcb_hbm.at[c, s, 0], sstage.at[0])
    pltpu.sync_copy(dstb_hbm.at[s, 0], dstage.at[0])
    plsc.subcore_barrier()

    def wait_s(b):
        pltpu.make_async_copy(gbuf.at[b], spmem_acc.at[dstage.at[0, 0]],
                              sems.at[b]).wait()

    def step(j, _):
        bi = j // CB
        ci = j % CB

        @pl.when((ci == SOFF) & (bi + 1 < NBLK))
        def _():
            slot = (bi + 1) % 2
            pltpu.sync_copy(srcb_hbm.at[c, s, bi + 1], sstage.at[slot])
            pltpu.sync_copy(dstb_hbm.at[s, bi + 1], dstage.at[slot])

        @pl.when(j < NCHUNK)
        def _():
            b = j % NB

            @pl.when(j >= NB)
            def _():
                wait_s(b)
            pltpu.async_copy(x2_hbm.at[sstage.at[bi % 2, ci]], gbuf.at[b],
                             semg.at[b])

        jj = j - PF

        @pl.when(jj >= 0)
        def _():
            b = jj % NB
            pltpu.make_async_copy(x2_hbm.at[sstage.at[0, 0]], gbuf.at[b],
                                  semg.at[b]).wait()
            pltpu.async_copy(gbuf.at[b],
                             spmem_acc.at[dstage.at[(jj // CB) % 2, jj % CB]],
                             sems.at[b], add=True)
        return 0
    lax.fori_loop(0, NCHUNK + PF, step, 0)

    def tail(b, _):
        wait_s(b)
        return 0
    lax.fori_loop(0, NB, tail, 0)

    plsc.subcore_barrier()

    def drain(k, _):
        r0 = s * ROWS_PT + k * ZROWS
        pltpu.sync_copy(spmem_acc.at[pl.ds(r0, ZROWS)],
                        out_hbm.at[c, pl.ds(r0, ZROWS)])
        return 0
    lax.fori_loop(0, ROWS_PT // ZROWS, drain, 0)


_agg128 = pl.kernel(
    _agg_body,
    out_type=jax.ShapeDtypeStruct((2, NPAD, HH), jnp.float32),
    mesh=_MESH,
    scratch_types=[
        pltpu.VMEM((2, CB, K), jnp.int32),
        pltpu.VMEM((2, CB, K), jnp.int32),
        pltpu.VMEM((NB, K, HH), jnp.float32),
        pltpu.VMEM((ZROWS, HH), jnp.float32),
        pltpu.VMEM_SHARED((NPAD, HH), jnp.float32),
        pltpu.SemaphoreType.DMA((NB,)),
        pltpu.SemaphoreType.DMA((NB,)),
    ],
    name="gcn_agg128",
)


BN = 1000


def _norms(dout, din):
    ns = jnp.where(dout > 0, lax.rsqrt(jnp.maximum(dout, 1.0)), 0.0)
    nd = jnp.where(din > 0, lax.rsqrt(jnp.maximum(din, 1.0)), 0.0)
    return ns, nd


def _tc0_body(x_ref, dout_ref, din_ref, w_ref, p_ref, ns_ref, nd_ref):
    ns, nd = _norms(dout_ref[...], din_ref[...])
    t = x_ref[...] * ns
    p = jnp.dot(t, w_ref[...], preferred_element_type=jnp.float32)
    p_ref[...] = jnp.stack([p[:, :HH], p[:, HH:]], axis=0)
    ns_ref[...] = ns
    nd_ref[...] = nd


def _tc0(x, dout, din, w1):
    return pl.pallas_call(
        _tc0_body,
        grid=(N // BN,),
        in_specs=[
            pl.BlockSpec((BN, IN_FEATS), lambda i: (i, 0)),
            pl.BlockSpec((BN, 1), lambda i: (i, 0)),
            pl.BlockSpec((BN, 1), lambda i: (i, 0)),
            pl.BlockSpec((IN_FEATS, H), lambda i: (0, 0)),
        ],
        out_specs=[
            pl.BlockSpec((2, BN, HH), lambda i: (0, i, 0)),
            pl.BlockSpec((BN, 1), lambda i: (i, 0)),
            pl.BlockSpec((BN, 1), lambda i: (i, 0)),
        ],
        out_shape=[
            jax.ShapeDtypeStruct((2, N, HH), jnp.float32),
            jax.ShapeDtypeStruct((N, 1), jnp.float32),
            jax.ShapeDtypeStruct((N, 1), jnp.float32),
        ],
    )(x, dout, din, w1)


def _mid_body(y_ref, b_ref, w_ref, ns_ref, nd_ref, p_ref):
    ns = ns_ref[...]
    nd = nd_ref[...]
    z0 = ns * jax.nn.leaky_relu(y_ref[0] * nd + b_ref[0, :, :HH], 0.01)
    z1 = ns * jax.nn.leaky_relu(y_ref[1] * nd + b_ref[0, :, HH:], 0.01)
    p = jnp.dot(z0, w_ref[0], preferred_element_type=jnp.float32)
    p += jnp.dot(z1, w_ref[1], preferred_element_type=jnp.float32)
    p_ref[...] = jnp.stack([p[:, :HH], p[:, HH:]], axis=0)


def _tc_mid(y, bvec, w2, ns, nd):
    return pl.pallas_call(
        _mid_body,
        grid=(N // BN,),
        in_specs=[
            pl.BlockSpec((2, BN, HH), lambda i: (0, i, 0)),
            pl.BlockSpec((1, 1, H), lambda i: (0, 0, 0)),
            pl.BlockSpec((2, HH, H), lambda i: (0, 0, 0)),
            pl.BlockSpec((BN, 1), lambda i: (i, 0)),
            pl.BlockSpec((BN, 1), lambda i: (i, 0)),
        ],
        out_specs=pl.BlockSpec((2, BN, HH), lambda i: (0, i, 0)),
        out_shape=jax.ShapeDtypeStruct((2, N, HH), jnp.float32),
    )(y, bvec, w2, ns, nd)


def _final_body(y_ref, b_ref, lw_ref, lb_ref, nd_ref, out_ref):
    nd = nd_ref[...]
    z0 = jax.nn.leaky_relu(y_ref[0] * nd + b_ref[0, :, :HH], 0.01)
    z1 = jax.nn.leaky_relu(y_ref[1] * nd + b_ref[0, :, HH:], 0.01)
    o = jnp.dot(z0, lw_ref[0], preferred_element_type=jnp.float32)
    o += jnp.dot(z1, lw_ref[1], preferred_element_type=jnp.float32)
    out_ref[...] = o + lb_ref[...]


def _tc_final(y, bvec, lin_w2, lin_b, nd):
    return pl.pallas_call(
        _final_body,
        grid=(N // BN,),
        in_specs=[
            pl.BlockSpec((2, BN, HH), lambda i: (0, i, 0)),
            pl.BlockSpec((1, 1, H), lambda i: (0, 0, 0)),
            pl.BlockSpec((2, HH, NUM_CLASSES), lambda i: (0, 0, 0)),
            pl.BlockSpec((1, NUM_CLASSES), lambda i: (0, 0)),
            pl.BlockSpec((BN, 1), lambda i: (i, 0)),
        ],
        out_specs=pl.BlockSpec((BN, NUM_CLASSES), lambda i: (i, 0)),
        out_shape=jax.ShapeDtypeStruct((N, NUM_CLASSES), jnp.float32),
    )(y, bvec, lin_w2, lin_b, nd)


def kernel(in_feat, edge_index, W1, b1, W2, b2, W3, b3, W4, b4, W5, b5,
           lin_W, lin_b):
    src = edge_index[0]
    dst = edge_index[1]
    srcb = src.reshape(NT, NBLK, CB, K)
    dstb = dst.reshape(NT, NBLK, CB, K)
    srcoff = jnp.stack([src, src + N]).reshape(2, NT, NBLK, CB, K)
    dstoff = jnp.stack([dst, dst + N]).reshape(2, NT, NBLK, CB, K)

    ones2n = jnp.ones((2 * N, HH), jnp.float32)
    din = _agg128(ones2n, srcoff, dstb)[0, :N, :1]
    dout = _agg128(ones2n, dstoff, srcb)[0, :N, :1]
    p, ns, nd = _tc0(in_feat, dout, din, W1)
    y = _agg128(p.reshape(2 * N, HH), srcoff, dstb)[:, :N, :]
    for b, W in ((b1, W2), (b2, W3), (b3, W4), (b4, W5)):
        p = _tc_mid(y, b.reshape(1, 1, H), W.reshape(2, HH, H), ns, nd)
        y = _agg128(p.reshape(2 * N, HH), srcoff, dstb)[:, :N, :]
    return _tc_final(y, b5.reshape(1, 1, H), lin_W.reshape(2, HH, NUM_CLASSES),
                     lin_b.reshape(1, NUM_CLASSES), nd)


# speedup vs baseline: 1.2382x; 1.2382x over previous
import jax
import jax.numpy as jnp
from jax import lax
from jax.experimental import pallas as pl
from jax.experimental.pallas import tpu as pltpu
from jax.experimental.pallas import tpu_sc as plsc

N = 10000
E = 320000
IN_FEATS = 128
H = 256
NUM_CLASSES = 64
HH = H // 2

NT = 16
NSC = 2
EPT = E // NT
K = 40
CB = 25
NBLK = EPT // (CB * K)
NPAD = 10240
ROWS_PT = NPAD // NT
ZROWS = 16

_MESH = plsc.VectorSubcoreMesh(core_axis_name="c", subcore_axis_name="s")


NCHUNK = EPT // K
NB = 4
PF = 2
SOFF = 4


def _agg_body(x2_hbm, srcb_hbm, dstb_hbm, out_hbm, sstage, dstage,
              gbuf, zbuf, spmem_acc, semg, sems, semt):
    c = lax.axis_index("c")
    s = lax.axis_index("s")
    zeros16 = jnp.zeros((16,), jnp.float32)

    def zrow(i, _):
        for j in range(HH // 16):
            zbuf[i, pl.ds(j * 16, 16)] = zeros16
        return 0
    lax.fori_loop(0, ZROWS, zrow, 0)

    def zcp(k, _):
        pltpu.sync_copy(zbuf, spmem_acc.at[pl.ds(s * ROWS_PT + k * ZROWS, ZROWS)])
        return 0
    lax.fori_loop(0, ROWS_PT // ZROWS, zcp, 0)

    pltpu.sync_copy(srcb_hbm.at[c, s, 0], sstage.at[0])
    pltpu.sync_copy(dstb_hbm.at[s, 0], dstage.at[0])
    plsc.subcore_barrier()

    def wait_stage(slot):
        pltpu.make_async_copy(srcb_hbm.at[c, s, 0], sstage.at[slot],
                              semt.at[slot]).wait()
        pltpu.make_async_copy(dstb_hbm.at[s, 0], dstage.at[slot],
                              semt.at[slot]).wait()

    def wait_s(b):
        pltpu.make_async_copy(gbuf.at[b], spmem_acc.at[dstage.at[0, 0]],
                              sems.at[b]).wait()

    def step(j, _):
        bi = j // CB
        ci = j % CB

        @pl.when((ci == SOFF) & (bi + 1 < NBLK))
        def _():
            slot = (bi + 1) % 2
            pltpu.async_copy(srcb_hbm.at[c, s, bi + 1], sstage.at[slot],
                             semt.at[slot])
            pltpu.async_copy(dstb_hbm.at[s, bi + 1], dstage.at[slot],
                             semt.at[slot])

        @pl.when((ci == 0) & (bi > 0) & (bi < NBLK))
        def _():
            wait_stage(bi % 2)

        @pl.when(j < NCHUNK)
        def _():
            b = j % NB

            @pl.when(j >= NB)
            def _():
                wait_s(b)
            pltpu.async_copy(x2_hbm.at[sstage.at[bi % 2, ci]], gbuf.at[b],
                             semg.at[b])

        jj = j - PF

        @pl.when(jj >= 0)
        def _():
            b = jj % NB
            pltpu.make_async_copy(x2_hbm.at[sstage.at[0, 0]], gbuf.at[b],
                                  semg.at[b]).wait()
            pltpu.async_copy(gbuf.at[b],
                             spmem_acc.at[dstage.at[(jj // CB) % 2, jj % CB]],
                             sems.at[b], add=True)
        return 0
    lax.fori_loop(0, NCHUNK + PF, step, 0)

    def tail(b, _):
        wait_s(b)
        return 0
    lax.fori_loop(0, NB, tail, 0)

    plsc.subcore_barrier()

    def drain(k, _):
        r0 = s * ROWS_PT + k * ZROWS
        pltpu.sync_copy(spmem_acc.at[pl.ds(r0, ZROWS)],
                        out_hbm.at[c, pl.ds(r0, ZROWS)])
        return 0
    lax.fori_loop(0, ROWS_PT // ZROWS, drain, 0)


_agg128 = pl.kernel(
    _agg_body,
    out_type=jax.ShapeDtypeStruct((2, NPAD, HH), jnp.float32),
    mesh=_MESH,
    scratch_types=[
        pltpu.VMEM((2, CB, K), jnp.int32),
        pltpu.VMEM((2, CB, K), jnp.int32),
        pltpu.VMEM((NB, K, HH), jnp.float32),
        pltpu.VMEM((ZROWS, HH), jnp.float32),
        pltpu.VMEM_SHARED((NPAD, HH), jnp.float32),
        pltpu.SemaphoreType.DMA((NB,)),
        pltpu.SemaphoreType.DMA((NB,)),
        pltpu.SemaphoreType.DMA((2,)),
    ],
    name="gcn_agg128",
)


BN = 1000


def _norms(dout, din):
    ns = jnp.where(dout > 0, lax.rsqrt(jnp.maximum(dout, 1.0)), 0.0)
    nd = jnp.where(din > 0, lax.rsqrt(jnp.maximum(din, 1.0)), 0.0)
    return ns, nd


def _tc0_body(x_ref, dout_ref, din_ref, w_ref, p_ref, ns_ref, nd_ref):
    ns, nd = _norms(dout_ref[...], din_ref[...])
    t = x_ref[...] * ns
    p = jnp.dot(t, w_ref[...], preferred_element_type=jnp.float32)
    p_ref[...] = jnp.stack([p[:, :HH], p[:, HH:]], axis=0)
    ns_ref[...] = ns
    nd_ref[...] = nd


def _tc0(x, dout, din, w1):
    return pl.pallas_call(
        _tc0_body,
        grid=(N // BN,),
        in_specs=[
            pl.BlockSpec((BN, IN_FEATS), lambda i: (i, 0)),
            pl.BlockSpec((BN, 1), lambda i: (i, 0)),
            pl.BlockSpec((BN, 1), lambda i: (i, 0)),
            pl.BlockSpec((IN_FEATS, H), lambda i: (0, 0)),
        ],
        out_specs=[
            pl.BlockSpec((2, BN, HH), lambda i: (0, i, 0)),
            pl.BlockSpec((BN, 1), lambda i: (i, 0)),
            pl.BlockSpec((BN, 1), lambda i: (i, 0)),
        ],
        out_shape=[
            jax.ShapeDtypeStruct((2, N, HH), jnp.float32),
            jax.ShapeDtypeStruct((N, 1), jnp.float32),
            jax.ShapeDtypeStruct((N, 1), jnp.float32),
        ],
    )(x, dout, din, w1)


def _mid_body(y_ref, b_ref, w_ref, ns_ref, nd_ref, p_ref):
    ns = ns_ref[...]
    nd = nd_ref[...]
    z0 = ns * jax.nn.leaky_relu(y_ref[0] * nd + b_ref[0, :, :HH], 0.01)
    z1 = ns * jax.nn.leaky_relu(y_ref[1] * nd + b_ref[0, :, HH:], 0.01)
    p = jnp.dot(z0, w_ref[0], preferred_element_type=jnp.float32)
    p += jnp.dot(z1, w_ref[1], preferred_element_type=jnp.float32)
    p_ref[...] = jnp.stack([p[:, :HH], p[:, HH:]], axis=0)


def _tc_mid(y, bvec, w2, ns, nd):
    return pl.pallas_call(
        _mid_body,
        grid=(N // BN,),
        in_specs=[
            pl.BlockSpec((2, BN, HH), lambda i: (0, i, 0)),
            pl.BlockSpec((1, 1, H), lambda i: (0, 0, 0)),
            pl.BlockSpec((2, HH, H), lambda i: (0, 0, 0)),
            pl.BlockSpec((BN, 1), lambda i: (i, 0)),
            pl.BlockSpec((BN, 1), lambda i: (i, 0)),
        ],
        out_specs=pl.BlockSpec((2, BN, HH), lambda i: (0, i, 0)),
        out_shape=jax.ShapeDtypeStruct((2, N, HH), jnp.float32),
    )(y, bvec, w2, ns, nd)


def _final_body(y_ref, b_ref, lw_ref, lb_ref, nd_ref, out_ref):
    nd = nd_ref[...]
    z0 = jax.nn.leaky_relu(y_ref[0] * nd + b_ref[0, :, :HH], 0.01)
    z1 = jax.nn.leaky_relu(y_ref[1] * nd + b_ref[0, :, HH:], 0.01)
    o = jnp.dot(z0, lw_ref[0], preferred_element_type=jnp.float32)
    o += jnp.dot(z1, lw_ref[1], preferred_element_type=jnp.float32)
    out_ref[...] = o + lb_ref[...]


def _tc_final(y, bvec, lin_w2, lin_b, nd):
    return pl.pallas_call(
        _final_body,
        grid=(N // BN,),
        in_specs=[
            pl.BlockSpec((2, BN, HH), lambda i: (0, i, 0)),
            pl.BlockSpec((1, 1, H), lambda i: (0, 0, 0)),
            pl.BlockSpec((2, HH, NUM_CLASSES), lambda i: (0, 0, 0)),
            pl.BlockSpec((1, NUM_CLASSES), lambda i: (0, 0)),
            pl.BlockSpec((BN, 1), lambda i: (i, 0)),
        ],
        out_specs=pl.BlockSpec((BN, NUM_CLASSES), lambda i: (i, 0)),
        out_shape=jax.ShapeDtypeStruct((N, NUM_CLASSES), jnp.float32),
    )(y, bvec, lin_w2, lin_b, nd)


def kernel(in_feat, edge_index, W1, b1, W2, b2, W3, b3, W4, b4, W5, b5,
           lin_W, lin_b):
    src = edge_index[0]
    dst = edge_index[1]
    srcb = src.reshape(NT, NBLK, CB, K)
    dstb = dst.reshape(NT, NBLK, CB, K)
    srcoff = jnp.stack([src, src + N]).reshape(2, NT, NBLK, CB, K)
    dstoff = jnp.stack([dst, dst + N]).reshape(2, NT, NBLK, CB, K)

    ones2n = jnp.ones((2 * N, HH), jnp.float32)
    din = _agg128(ones2n, srcoff, dstb)[0, :N, :1]
    dout = _agg128(ones2n, dstoff, srcb)[0, :N, :1]
    p, ns, nd = _tc0(in_feat, dout, din, W1)
    y = _agg128(p.reshape(2 * N, HH), srcoff, dstb)[:, :N, :]
    for b, W in ((b1, W2), (b2, W3), (b3, W4), (b4, W5)):
        p = _tc_mid(y, b.reshape(1, 1, H), W.reshape(2, HH, H), ns, nd)
        y = _agg128(p.reshape(2 * N, HH), srcoff, dstb)[:, :N, :]
    return _tc_final(y, b5.reshape(1, 1, H), lin_W.reshape(2, HH, NUM_CLASSES),
                     lin_b.reshape(1, NUM_CLASSES), nd)


# speedup vs baseline: 1.3537x; 1.0932x over previous
import jax
import jax.numpy as jnp
from jax import lax
from jax.experimental import pallas as pl
from jax.experimental.pallas import tpu as pltpu
from jax.experimental.pallas import tpu_sc as plsc

N = 10000
E = 320000
IN_FEATS = 128
H = 256
NUM_CLASSES = 64
HH = H // 2

NT = 16
NSC = 2
EPT = E // NT
K = 40
CB = 25
NBLK = EPT // (CB * K)
NPAD = 10240
ROWS_PT = NPAD // NT
ZROWS = 16

_MESH = plsc.VectorSubcoreMesh(core_axis_name="c", subcore_axis_name="s")


NCHUNK = EPT // K
NB = 4
PF = 2
SOFF = 4


def _agg_body(x2_hbm, srcb_hbm, dstb_hbm, out_hbm, sstage, dstage,
              gbuf, zbuf, spmem_acc, semg, sems, semt, semz):
    c = lax.axis_index("c")
    s = lax.axis_index("s")
    zeros16 = jnp.zeros((16,), jnp.float32)

    def zrow(i, _):
        for j in range(HH // 16):
            zbuf[i, pl.ds(j * 16, 16)] = zeros16
        return 0
    lax.fori_loop(0, ZROWS, zrow, 0)

    def zcp(k, _):
        pltpu.async_copy(zbuf, spmem_acc.at[pl.ds(s * ROWS_PT + k * ZROWS, ZROWS)],
                         semz)
        return 0
    lax.fori_loop(0, ROWS_PT // ZROWS, zcp, 0)

    def zwait(k, _):
        pltpu.make_async_copy(zbuf, spmem_acc.at[pl.ds(s * ROWS_PT, ZROWS)],
                              semz).wait()
        return 0
    lax.fori_loop(0, ROWS_PT // ZROWS, zwait, 0)

    pltpu.sync_copy(srcb_hbm.at[c, s, 0], sstage.at[0])
    pltpu.sync_copy(dstb_hbm.at[s, 0], dstage.at[0])
    plsc.subcore_barrier()

    def wait_stage(slot):
        pltpu.make_async_copy(srcb_hbm.at[c, s, 0], sstage.at[slot],
                              semt.at[slot]).wait()
        pltpu.make_async_copy(dstb_hbm.at[s, 0], dstage.at[slot],
                              semt.at[slot]).wait()

    def wait_s(b):
        pltpu.make_async_copy(gbuf.at[b], spmem_acc.at[dstage.at[0, 0]],
                              sems.at[b]).wait()

    def step(j, _):
        bi = j // CB
        ci = j % CB

        @pl.when((ci == SOFF) & (bi + 1 < NBLK))
        def _():
            slot = (bi + 1) % 2
            pltpu.async_copy(srcb_hbm.at[c, s, bi + 1], sstage.at[slot],
                             semt.at[slot])
            pltpu.async_copy(dstb_hbm.at[s, bi + 1], dstage.at[slot],
                             semt.at[slot])

        @pl.when((ci == 0) & (bi > 0) & (bi < NBLK))
        def _():
            wait_stage(bi % 2)

        @pl.when(j < NCHUNK)
        def _():
            b = j % NB

            @pl.when(j >= NB)
            def _():
                wait_s(b)
            pltpu.async_copy(x2_hbm.at[sstage.at[bi % 2, ci]], gbuf.at[b],
                             semg.at[b])

        jj = j - PF

        @pl.when(jj >= 0)
        def _():
            b = jj % NB
            pltpu.make_async_copy(x2_hbm.at[sstage.at[0, 0]], gbuf.at[b],
                                  semg.at[b]).wait()
            pltpu.async_copy(gbuf.at[b],
                             spmem_acc.at[dstage.at[(jj // CB) % 2, jj % CB]],
                             sems.at[b], add=True)
        return 0
    lax.fori_loop(0, NCHUNK + PF, step, 0)

    def tail(b, _):
        wait_s(b)
        return 0
    lax.fori_loop(0, NB, tail, 0)

    plsc.subcore_barrier()

    def drain(k, _):
        r0 = s * ROWS_PT + k * ZROWS
        pltpu.async_copy(spmem_acc.at[pl.ds(r0, ZROWS)],
                         out_hbm.at[c, pl.ds(r0, ZROWS)], semz)
        return 0
    lax.fori_loop(0, ROWS_PT // ZROWS, drain, 0)

    def dwait(k, _):
        pltpu.make_async_copy(spmem_acc.at[pl.ds(s * ROWS_PT, ZROWS)],
                              out_hbm.at[c, pl.ds(s * ROWS_PT, ZROWS)],
                              semz).wait()
        return 0
    lax.fori_loop(0, ROWS_PT // ZROWS, dwait, 0)


_agg128 = pl.kernel(
    _agg_body,
    out_type=jax.ShapeDtypeStruct((2, NPAD, HH), jnp.float32),
    mesh=_MESH,
    scratch_types=[
        pltpu.VMEM((2, CB, K), jnp.int32),
        pltpu.VMEM((2, CB, K), jnp.int32),
        pltpu.VMEM((NB, K, HH), jnp.float32),
        pltpu.VMEM((ZROWS, HH), jnp.float32),
        pltpu.VMEM_SHARED((NPAD, HH), jnp.float32),
        pltpu.SemaphoreType.DMA((NB,)),
        pltpu.SemaphoreType.DMA((NB,)),
        pltpu.SemaphoreType.DMA((2,)),
        pltpu.SemaphoreType.DMA,
    ],
    name="gcn_agg128",
)


BN = 1000


def _norms(dout, din):
    ns = jnp.where(dout > 0, lax.rsqrt(jnp.maximum(dout, 1.0)), 0.0)
    nd = jnp.where(din > 0, lax.rsqrt(jnp.maximum(din, 1.0)), 0.0)
    return ns, nd


def _tc0_body(x_ref, dout_ref, din_ref, w_ref, p_ref, ns_ref, nd_ref):
    ns, nd = _norms(dout_ref[...], din_ref[...])
    t = x_ref[...] * ns
    p = jnp.dot(t, w_ref[...], preferred_element_type=jnp.float32)
    p_ref[...] = jnp.stack([p[:, :HH], p[:, HH:]], axis=0)
    ns_ref[...] = ns
    nd_ref[...] = nd


def _tc0(x, dout, din, w1):
    return pl.pallas_call(
        _tc0_body,
        grid=(N // BN,),
        in_specs=[
            pl.BlockSpec((BN, IN_FEATS), lambda i: (i, 0)),
            pl.BlockSpec((BN, 1), lambda i: (i, 0)),
            pl.BlockSpec((BN, 1), lambda i: (i, 0)),
            pl.BlockSpec((IN_FEATS, H), lambda i: (0, 0)),
        ],
        out_specs=[
            pl.BlockSpec((2, BN, HH), lambda i: (0, i, 0)),
            pl.BlockSpec((BN, 1), lambda i: (i, 0)),
            pl.BlockSpec((BN, 1), lambda i: (i, 0)),
        ],
        out_shape=[
            jax.ShapeDtypeStruct((2, N, HH), jnp.float32),
            jax.ShapeDtypeStruct((N, 1), jnp.float32),
            jax.ShapeDtypeStruct((N, 1), jnp.float32),
        ],
    )(x, dout, din, w1)


def _mid_body(y_ref, b_ref, w_ref, ns_ref, nd_ref, p_ref):
    ns = ns_ref[...]
    nd = nd_ref[...]
    z0 = ns * jax.nn.leaky_relu(y_ref[0] * nd + b_ref[0, :, :HH], 0.01)
    z1 = ns * jax.nn.leaky_relu(y_ref[1] * nd + b_ref[0, :, HH:], 0.01)
    p = jnp.dot(z0, w_ref[0], preferred_element_type=jnp.float32)
    p += jnp.dot(z1, w_ref[1], preferred_element_type=jnp.float32)
    p_ref[...] = jnp.stack([p[:, :HH], p[:, HH:]], axis=0)


def _tc_mid(y, bvec, w2, ns, nd):
    return pl.pallas_call(
        _mid_body,
        grid=(N // BN,),
        in_specs=[
            pl.BlockSpec((2, BN, HH), lambda i: (0, i, 0)),
            pl.BlockSpec((1, 1, H), lambda i: (0, 0, 0)),
            pl.BlockSpec((2, HH, H), lambda i: (0, 0, 0)),
            pl.BlockSpec((BN, 1), lambda i: (i, 0)),
            pl.BlockSpec((BN, 1), lambda i: (i, 0)),
        ],
        out_specs=pl.BlockSpec((2, BN, HH), lambda i: (0, i, 0)),
        out_shape=jax.ShapeDtypeStruct((2, N, HH), jnp.float32),
    )(y, bvec, w2, ns, nd)


def _final_body(y_ref, b_ref, lw_ref, lb_ref, nd_ref, out_ref):
    nd = nd_ref[...]
    z0 = jax.nn.leaky_relu(y_ref[0] * nd + b_ref[0, :, :HH], 0.01)
    z1 = jax.nn.leaky_relu(y_ref[1] * nd + b_ref[0, :, HH:], 0.01)
    o = jnp.dot(z0, lw_ref[0], preferred_element_type=jnp.float32)
    o += jnp.dot(z1, lw_ref[1], preferred_element_type=jnp.float32)
    out_ref[...] = o + lb_ref[...]


def _tc_final(y, bvec, lin_w2, lin_b, nd):
    return pl.pallas_call(
        _final_body,
        grid=(N // BN,),
        in_specs=[
            pl.BlockSpec((2, BN, HH), lambda i: (0, i, 0)),
            pl.BlockSpec((1, 1, H), lambda i: (0, 0, 0)),
            pl.BlockSpec((2, HH, NUM_CLASSES), lambda i: (0, 0, 0)),
            pl.BlockSpec((1, NUM_CLASSES), lambda i: (0, 0)),
            pl.BlockSpec((BN, 1), lambda i: (i, 0)),
        ],
        out_specs=pl.BlockSpec((BN, NUM_CLASSES), lambda i: (i, 0)),
        out_shape=jax.ShapeDtypeStruct((N, NUM_CLASSES), jnp.float32),
    )(y, bvec, lin_w2, lin_b, nd)


def kernel(in_feat, edge_index, W1, b1, W2, b2, W3, b3, W4, b4, W5, b5,
           lin_W, lin_b):
    src = edge_index[0]
    dst = edge_index[1]
    srcb = src.reshape(NT, NBLK, CB, K)
    dstb = dst.reshape(NT, NBLK, CB, K)
    srcoff = jnp.stack([src, src + N]).reshape(2, NT, NBLK, CB, K)
    dstoff = jnp.stack([dst, dst + N]).reshape(2, NT, NBLK, CB, K)

    ones2n = jnp.ones((2 * N, HH), jnp.float32)
    din = _agg128(ones2n, srcoff, dstb)[0, :N, :1]
    dout = _agg128(ones2n, dstoff, srcb)[0, :N, :1]
    p, ns, nd = _tc0(in_feat, dout, din, W1)
    y = _agg128(p.reshape(2 * N, HH), srcoff, dstb)[:, :N, :]
    for b, W in ((b1, W2), (b2, W3), (b3, W4), (b4, W5)):
        p = _tc_mid(y, b.reshape(1, 1, H), W.reshape(2, HH, H), ns, nd)
        y = _agg128(p.reshape(2 * N, HH), srcoff, dstb)[:, :N, :]
    return _tc_final(y, b5.reshape(1, 1, H), lin_W.reshape(2, HH, NUM_CLASSES),
                     lin_b.reshape(1, NUM_CLASSES), nd)


# speedup vs baseline: 1.4305x; 1.0568x over previous
import jax
import jax.numpy as jnp
from jax import lax
from jax.experimental import pallas as pl
from jax.experimental.pallas import tpu as pltpu
from jax.experimental.pallas import tpu_sc as plsc

N = 10000
E = 320000
IN_FEATS = 128
H = 256
NUM_CLASSES = 64
HH = H // 2

NT = 16
NSC = 2
EPT = E // NT
K = 50
CB = 25
NBLK = EPT // (CB * K)
NPAD = 10240
ROWS_PT = NPAD // NT
ZROWS = 16

_MESH = plsc.VectorSubcoreMesh(core_axis_name="c", subcore_axis_name="s")


NCHUNK = EPT // K
NB = 4
PF = 2
SOFF = 4


def _agg_body(x2_hbm, srcb_hbm, dstb_hbm, out_hbm, sstage, dstage,
              gbuf, zbuf, spmem_acc, semg, sems, semt, semz):
    c = lax.axis_index("c")
    s = lax.axis_index("s")
    zeros16 = jnp.zeros((16,), jnp.float32)

    def zrow(i, _):
        for j in range(HH // 16):
            zbuf[i, pl.ds(j * 16, 16)] = zeros16
        return 0
    lax.fori_loop(0, ZROWS, zrow, 0)

    def zcp(k, _):
        pltpu.async_copy(zbuf, spmem_acc.at[pl.ds(s * ROWS_PT + k * ZROWS, ZROWS)],
                         semz)
        return 0
    lax.fori_loop(0, ROWS_PT // ZROWS, zcp, 0)

    def zwait(k, _):
        pltpu.make_async_copy(zbuf, spmem_acc.at[pl.ds(s * ROWS_PT, ZROWS)],
                              semz).wait()
        return 0
    lax.fori_loop(0, ROWS_PT // ZROWS, zwait, 0)

    pltpu.sync_copy(srcb_hbm.at[c, s, 0], sstage.at[0])
    pltpu.sync_copy(dstb_hbm.at[s, 0], dstage.at[0])
    plsc.subcore_barrier()

    def wait_stage(slot):
        pltpu.make_async_copy(srcb_hbm.at[c, s, 0], sstage.at[slot],
                              semt.at[slot]).wait()
        pltpu.make_async_copy(dstb_hbm.at[s, 0], dstage.at[slot],
                              semt.at[slot]).wait()

    def wait_s(b):
        pltpu.make_async_copy(gbuf.at[b], spmem_acc.at[dstage.at[0, 0]],
                              sems.at[b]).wait()

    def step(j, _):
        bi = j // CB
        ci = j % CB

        @pl.when((ci == SOFF) & (bi + 1 < NBLK))
        def _():
            slot = (bi + 1) % 2
            pltpu.async_copy(srcb_hbm.at[c, s, bi + 1], sstage.at[slot],
                             semt.at[slot])
            pltpu.async_copy(dstb_hbm.at[s, bi + 1], dstage.at[slot],
                             semt.at[slot])

        @pl.when((ci == 0) & (bi > 0) & (bi < NBLK))
        def _():
            wait_stage(bi % 2)

        @pl.when(j < NCHUNK)
        def _():
            b = j % NB

            @pl.when(j >= NB)
            def _():
                wait_s(b)
            pltpu.async_copy(x2_hbm.at[sstage.at[bi % 2, ci]], gbuf.at[b],
                             semg.at[b])

        jj = j - PF

        @pl.when(jj >= 0)
        def _():
            b = jj % NB
            pltpu.make_async_copy(x2_hbm.at[sstage.at[0, 0]], gbuf.at[b],
                                  semg.at[b]).wait()
            pltpu.async_copy(gbuf.at[b],
                             spmem_acc.at[dstage.at[(jj // CB) % 2, jj % CB]],
                             sems.at[b], add=True)
        return 0
    lax.fori_loop(0, NCHUNK + PF, step, 0)

    def tail(b, _):
        wait_s(b)
        return 0
    lax.fori_loop(0, NB, tail, 0)

    plsc.subcore_barrier()

    def drain(k, _):
        r0 = s * ROWS_PT + k * ZROWS
        pltpu.async_copy(spmem_acc.at[pl.ds(r0, ZROWS)],
                         out_hbm.at[c, pl.ds(r0, ZROWS)], semz)
        return 0
    lax.fori_loop(0, ROWS_PT // ZROWS, drain, 0)

    def dwait(k, _):
        pltpu.make_async_copy(spmem_acc.at[pl.ds(s * ROWS_PT, ZROWS)],
                              out_hbm.at[c, pl.ds(s * ROWS_PT, ZROWS)],
                              semz).wait()
        return 0
    lax.fori_loop(0, ROWS_PT // ZROWS, dwait, 0)


_agg128 = pl.kernel(
    _agg_body,
    out_type=jax.ShapeDtypeStruct((2, NPAD, HH), jnp.float32),
    mesh=_MESH,
    scratch_types=[
        pltpu.VMEM((2, CB, K), jnp.int32),
        pltpu.VMEM((2, CB, K), jnp.int32),
        pltpu.VMEM((NB, K, HH), jnp.float32),
        pltpu.VMEM((ZROWS, HH), jnp.float32),
        pltpu.VMEM_SHARED((NPAD, HH), jnp.float32),
        pltpu.SemaphoreType.DMA((NB,)),
        pltpu.SemaphoreType.DMA((NB,)),
        pltpu.SemaphoreType.DMA((2,)),
        pltpu.SemaphoreType.DMA,
    ],
    name="gcn_agg128",
)


BN = 1000


def _norms(dout, din):
    ns = jnp.where(dout > 0, lax.rsqrt(jnp.maximum(dout, 1.0)), 0.0)
    nd = jnp.where(din > 0, lax.rsqrt(jnp.maximum(din, 1.0)), 0.0)
    return ns, nd


def _tc0_body(x_ref, dout_ref, din_ref, w_ref, p_ref, ns_ref, nd_ref):
    ns, nd = _norms(dout_ref[...], din_ref[...])
    t = x_ref[...] * ns
    p = jnp.dot(t, w_ref[...], preferred_element_type=jnp.float32)
    p_ref[...] = jnp.stack([p[:, :HH], p[:, HH:]], axis=0)
    ns_ref[...] = ns
    nd_ref[...] = nd


def _tc0(x, dout, din, w1):
    return pl.pallas_call(
        _tc0_body,
        grid=(N // BN,),
        in_specs=[
            pl.BlockSpec((BN, IN_FEATS), lambda i: (i, 0)),
            pl.BlockSpec((BN, 1), lambda i: (i, 0)),
            pl.BlockSpec((BN, 1), lambda i: (i, 0)),
            pl.BlockSpec((IN_FEATS, H), lambda i: (0, 0)),
        ],
        out_specs=[
            pl.BlockSpec((2, BN, HH), lambda i: (0, i, 0)),
            pl.BlockSpec((BN, 1), lambda i: (i, 0)),
            pl.BlockSpec((BN, 1), lambda i: (i, 0)),
        ],
        out_shape=[
            jax.ShapeDtypeStruct((2, N, HH), jnp.float32),
            jax.ShapeDtypeStruct((N, 1), jnp.float32),
            jax.ShapeDtypeStruct((N, 1), jnp.float32),
        ],
    )(x, dout, din, w1)


def _mid_body(y_ref, b_ref, w_ref, ns_ref, nd_ref, p_ref):
    ns = ns_ref[...]
    nd = nd_ref[...]
    z0 = ns * jax.nn.leaky_relu(y_ref[0] * nd + b_ref[0, :, :HH], 0.01)
    z1 = ns * jax.nn.leaky_relu(y_ref[1] * nd + b_ref[0, :, HH:], 0.01)
    p = jnp.dot(z0, w_ref[0], preferred_element_type=jnp.float32)
    p += jnp.dot(z1, w_ref[1], preferred_element_type=jnp.float32)
    p_ref[...] = jnp.stack([p[:, :HH], p[:, HH:]], axis=0)


def _tc_mid(y, bvec, w2, ns, nd):
    return pl.pallas_call(
        _mid_body,
        grid=(N // BN,),
        in_specs=[
            pl.BlockSpec((2, BN, HH), lambda i: (0, i, 0)),
            pl.BlockSpec((1, 1, H), lambda i: (0, 0, 0)),
            pl.BlockSpec((2, HH, H), lambda i: (0, 0, 0)),
            pl.BlockSpec((BN, 1), lambda i: (i, 0)),
            pl.BlockSpec((BN, 1), lambda i: (i, 0)),
        ],
        out_specs=pl.BlockSpec((2, BN, HH), lambda i: (0, i, 0)),
        out_shape=jax.ShapeDtypeStruct((2, N, HH), jnp.float32),
    )(y, bvec, w2, ns, nd)


def _final_body(y_ref, b_ref, lw_ref, lb_ref, nd_ref, out_ref):
    nd = nd_ref[...]
    z0 = jax.nn.leaky_relu(y_ref[0] * nd + b_ref[0, :, :HH], 0.01)
    z1 = jax.nn.leaky_relu(y_ref[1] * nd + b_ref[0, :, HH:], 0.01)
    o = jnp.dot(z0, lw_ref[0], preferred_element_type=jnp.float32)
    o += jnp.dot(z1, lw_ref[1], preferred_element_type=jnp.float32)
    out_ref[...] = o + lb_ref[...]


def _tc_final(y, bvec, lin_w2, lin_b, nd):
    return pl.pallas_call(
        _final_body,
        grid=(N // BN,),
        in_specs=[
            pl.BlockSpec((2, BN, HH), lambda i: (0, i, 0)),
            pl.BlockSpec((1, 1, H), lambda i: (0, 0, 0)),
            pl.BlockSpec((2, HH, NUM_CLASSES), lambda i: (0, 0, 0)),
            pl.BlockSpec((1, NUM_CLASSES), lambda i: (0, 0)),
            pl.BlockSpec((BN, 1), lambda i: (i, 0)),
        ],
        out_specs=pl.BlockSpec((BN, NUM_CLASSES), lambda i: (i, 0)),
        out_shape=jax.ShapeDtypeStruct((N, NUM_CLASSES), jnp.float32),
    )(y, bvec, lin_w2, lin_b, nd)


def kernel(in_feat, edge_index, W1, b1, W2, b2, W3, b3, W4, b4, W5, b5,
           lin_W, lin_b):
    src = edge_index[0]
    dst = edge_index[1]
    srcb = src.reshape(NT, NBLK, CB, K)
    dstb = dst.reshape(NT, NBLK, CB, K)
    srcoff = jnp.stack([src, src + N]).reshape(2, NT, NBLK, CB, K)
    dstoff = jnp.stack([dst, dst + N]).reshape(2, NT, NBLK, CB, K)

    ones2n = jnp.ones((2 * N, HH), jnp.float32)
    din = _agg128(ones2n, srcoff, dstb)[0, :N, :1]
    dout = _agg128(ones2n, dstoff, srcb)[0, :N, :1]
    p, ns, nd = _tc0(in_feat, dout, din, W1)
    y = _agg128(p.reshape(2 * N, HH), srcoff, dstb)[:, :N, :]
    for b, W in ((b1, W2), (b2, W3), (b3, W4), (b4, W5)):
        p = _tc_mid(y, b.reshape(1, 1, H), W.reshape(2, HH, H), ns, nd)
        y = _agg128(p.reshape(2 * N, HH), srcoff, dstb)[:, :N, :]
    return _tc_final(y, b5.reshape(1, 1, H), lin_W.reshape(2, HH, NUM_CLASSES),
                     lin_b.reshape(1, NUM_CLASSES), nd)
